# baseline (speedup 1.0000x reference)
import jax
import jax.numpy as jnp
from jax import lax
from jax.experimental import pallas as pl
from jax.experimental.pallas import tpu as pltpu
from jax.experimental.pallas import tpu_sc as plsc

NC, NS, L = 2, 16, 16
NW = NC * NS
HID = 128
ROWS = 65
NF = 4
B, S = 4096, 200
BG = B // 128
SLAB = NF * 128
WS = 40
NWIN = S // WS
WROW = WS * NF
STGST = WROW + 1
NRING = 4
TAB_WORDS = NF * ROWS * HID


def _body(ids_hbm, tab_hbm, out_hbm, tab_v, slab_v, stg_v, out_v, sem_tab,
          sem_ids, sem_out):
    wid = lax.axis_index("s") * NC + lax.axis_index("c")
    bg = wid
    b0 = bg * 128

    pltpu.async_copy(tab_hbm, tab_v, sem_tab).wait()

    iota = jnp.arange(L, dtype=jnp.int32)
    iotastg = iota * STGST

    def load_slabs(w, slot):
        return pltpu.async_copy(
            ids_hbm.at[pl.ds(w * WS, WS), bg], slab_v.at[slot], sem_ids)

    load_slabs(0, 0).wait()

    def win_body(w, _):
        slot = lax.rem(w, 2)
        s0 = w * WS

        @pl.when(w + 1 < NWIN)
        def _():
            load_slabs(w + 1, 1 - slot)

        @plsc.parallel_loop(0, WS, unroll=2)
        def tr_body(sl):
            for grp in range(128 // L):
                for f in range(NF):
                    v = slab_v[slot, sl, pl.ds(f * 128 + grp * L, L)]
                    plsc.store_scatter(
                        stg_v,
                        [iotastg + (grp * (L * STGST) + sl * NF + f)], v)

        def row_body(bl, _):
            ring = lax.rem(bl, NRING)

            @pl.when(bl >= NRING)
            def _():
                pltpu.make_async_copy(
                    out_v.at[0], out_hbm.at[0, pl.ds(0, WS)],
                    sem_out).wait()

            rbase = bl * STGST
            for q in range(WROW // L):
                vec = plsc.load_gather(stg_v, [iota + (rbase + q * L)])
                for j in range(4):
                    t = q * 4 + j
                    for c in range(HID // L):
                        t0 = tab_v[pl.ds(vec[4 * j + 0] + c * L, L)]
                        t1 = tab_v[pl.ds(vec[4 * j + 1] + c * L, L)]
                        t2 = tab_v[pl.ds(vec[4 * j + 2] + c * L, L)]
                        t3 = tab_v[pl.ds(vec[4 * j + 3] + c * L, L)]
                        out_v[ring, t, pl.ds(c * L, L)] = \
                            (t0 + t1) + (t2 + t3)

            pltpu.async_copy(out_v.at[ring],
                             out_hbm.at[b0 + bl, pl.ds(s0, WS)], sem_out)
            return 0

        lax.fori_loop(0, 128, row_body, 0)

        for _ in range(NRING):
            pltpu.make_async_copy(out_v.at[0], out_hbm.at[0, pl.ds(0, WS)],
                                  sem_out).wait()

        @pl.when(w + 1 < NWIN)
        def _():
            pltpu.make_async_copy(
                slab_v.at[0], ids_hbm.at[pl.ds(0, WS), 0], sem_ids).wait()
        return 0

    lax.fori_loop(0, NWIN, win_body, 0)


@jax.jit
def _run(offs, tab_flat):
    mesh = plsc.VectorSubcoreMesh(core_axis_name="c", subcore_axis_name="s",
                                  num_cores=NC, num_subcores=NS)
    return pl.kernel(
        _body,
        out_type=jax.ShapeDtypeStruct((B, S, HID), jnp.float32),
        mesh=mesh,
        scratch_types=[
            pltpu.VMEM((TAB_WORDS,), jnp.float32),
            pltpu.VMEM((2, WS, SLAB), jnp.int32),
            pltpu.VMEM((128 * STGST,), jnp.int32),
            pltpu.VMEM((NRING, WS, HID), jnp.float32),
            pltpu.SemaphoreType.DMA,
            pltpu.SemaphoreType.DMA,
            pltpu.SemaphoreType.DMA,
        ],
        compiler_params=pltpu.CompilerParams(needs_layout_passes=False),
    )(offs, tab_flat)


def kernel(noise_ids, W0, W1, W2, W3):
    featbase = jnp.array([i * ROWS * HID for i in range(NF)], jnp.int32)
    offs = noise_ids * HID + featbase
    offs_sb = (offs.reshape(BG, 128, S, NF)
               .transpose(2, 0, 3, 1)
               .reshape(S, BG, SLAB))
    tab_flat = jnp.concatenate([W0, W1, W2, W3], axis=0).reshape(-1)
    return _run(offs_sb, tab_flat)


# speedup vs baseline: 3.4757x; 3.4757x over previous
import jax
import jax.numpy as jnp
from jax import lax
from jax.experimental import pallas as pl
from jax.experimental.pallas import tpu as pltpu
from jax.experimental.pallas import tpu_sc as plsc

NC, NS, L = 2, 16, 16
NW = NC * NS
HID = 128
ROWS = 65
NF = 4
B, S = 4096, 200
BG = B // 128
SLAB = NF * 128
WS = 40
NWIN = S // WS
WROW = WS * NF
STGST = WROW + 1
NRING = 4
TAB_WORDS = NF * ROWS * HID


def _body(ids_hbm, tab_hbm, out_hbm, tab_v, slab_v, stg_v, out_v, sem_tab,
          sem_ids, sem_out):
    wid = lax.axis_index("s") * NC + lax.axis_index("c")
    bg = wid
    b0 = bg * 128

    pltpu.async_copy(tab_hbm, tab_v, sem_tab).wait()

    iota = jnp.arange(L, dtype=jnp.int32)
    iotastg = iota * STGST

    def win_body(w, _):
        s0 = w * WS
        pltpu.async_copy(ids_hbm.at[pl.ds(w * WS, WS), bg], slab_v,
                         sem_ids).wait()

        @plsc.parallel_loop(0, WS, unroll=2)
        def tr_body(sl):
            for grp in range(128 // L):
                for f in range(NF):
                    v = slab_v[sl, pl.ds(f * 128 + grp * L, L)]
                    plsc.store_scatter(
                        stg_v,
                        [iotastg + (grp * (L * STGST) + sl * NF + f)], v)

        def quad_body(qd, _):
            ring = lax.rem(qd, 2)

            @pl.when(qd >= 2)
            def _():
                pltpu.make_async_copy(
                    out_v.at[0], out_hbm.at[pl.ds(0, 4), pl.ds(0, WS)],
                    sem_out).wait()

            @plsc.parallel_loop(0, 4 * (WROW // L), unroll=2)
            def sub_body(i):
                bl = i % 4
                q = i // 4
                vec = plsc.load_gather(
                    stg_v,
                    [iota + ((qd * 4 + bl) * STGST + q * L)])
                for j in range(4):
                    t = q * 4 + j
                    for c in range(HID // L):
                        t0 = tab_v[pl.ds(vec[4 * j + 0] + c * L, L)]
                        t1 = tab_v[pl.ds(vec[4 * j + 1] + c * L, L)]
                        t2 = tab_v[pl.ds(vec[4 * j + 2] + c * L, L)]
                        t3 = tab_v[pl.ds(vec[4 * j + 3] + c * L, L)]
                        out_v[ring, bl, t, pl.ds(c * L, L)] = \
                            (t0 + t1) + (t2 + t3)

            for bl in range(4):
                pltpu.async_copy(
                    out_v.at[ring, bl],
                    out_hbm.at[b0 + qd * 4 + bl, pl.ds(s0, WS)], sem_out)
            return 0

        lax.fori_loop(0, 32, quad_body, 0)

        for _ in range(2):
            pltpu.make_async_copy(
                out_v.at[0], out_hbm.at[pl.ds(0, 4), pl.ds(0, WS)],
                sem_out).wait()
        return 0

    lax.fori_loop(0, NWIN, win_body, 0)


@jax.jit
def _run(offs, tab_flat):
    mesh = plsc.VectorSubcoreMesh(core_axis_name="c", subcore_axis_name="s",
                                  num_cores=NC, num_subcores=NS)
    return pl.kernel(
        _body,
        out_type=jax.ShapeDtypeStruct((B, S, HID), jnp.float32),
        mesh=mesh,
        scratch_types=[
            pltpu.VMEM((TAB_WORDS,), jnp.float32),
            pltpu.VMEM((WS, SLAB), jnp.int32),
            pltpu.VMEM((128 * STGST,), jnp.int32),
            pltpu.VMEM((2, 4, WS, HID), jnp.float32),
            pltpu.SemaphoreType.DMA,
            pltpu.SemaphoreType.DMA,
            pltpu.SemaphoreType.DMA,
        ],
        compiler_params=pltpu.CompilerParams(needs_layout_passes=False),
    )(offs, tab_flat)


def kernel(noise_ids, W0, W1, W2, W3):
    featbase = jnp.array([i * ROWS * HID for i in range(NF)], jnp.int32)
    offs = noise_ids * HID + featbase
    offs_sb = (offs.reshape(BG, 128, S, NF)
               .transpose(2, 0, 3, 1)
               .reshape(S, BG, SLAB))
    tab_flat = jnp.concatenate([W0, W1, W2, W3], axis=0).reshape(-1)
    return _run(offs_sb, tab_flat)
